# baseline (speedup 1.0000x reference)
import functools

import jax
import jax.numpy as jnp
from jax import lax
from jax.experimental import pallas as pl
from jax.experimental.pallas import tpu as pltpu
from jax.experimental.pallas import tpu_sc as plsc

_LANES = 16
_NBUF = 4


def _make_sc_kernel(n_rows, vocab, embed, seq_len):
    n_workers = 32
    assert n_rows % (n_workers * seq_len) == 0
    rows_per_w = n_rows // n_workers
    chunk = seq_len
    n_chunks = rows_per_w // chunk
    n_pairs = n_chunks // 2
    n_outer = n_pairs // 2
    assert n_pairs % 2 == 0 and chunk % 8 == 0
    vregs_per_row = embed // _LANES
    seg = [0, chunk // 3 // 8 * 8, 2 * (chunk // 3) // 8 * 8, chunk]

    mesh = plsc.VectorSubcoreMesh(core_axis_name="c", subcore_axis_name="s")

    @functools.partial(
        pl.kernel,
        mesh=mesh,
        out_type=jax.ShapeDtypeStruct((n_rows, embed), jnp.float32),
        scratch_types=[
            [pltpu.VMEM((chunk,), jnp.int32) for _ in range(_NBUF)],
            [pltpu.VMEM((chunk, embed), jnp.float32) for _ in range(_NBUF)],
            pltpu.VMEM((seq_len, embed), jnp.float32),
            [pltpu.SemaphoreType.DMA for _ in range(_NBUF)],
            [pltpu.SemaphoreType.DMA for _ in range(_NBUF)],
        ],
    )
    def sc_kernel(idx_hbm, table_hbm, pe_hbm, out_hbm, idx, rows, pe_v, gsem, osem):
        wid = lax.axis_index("s") * 2 + lax.axis_index("c")
        row0 = wid * rows_per_w
        coef = jnp.float32(1.0 / (embed**0.5))

        pltpu.sync_copy(pe_hbm, pe_v)

        def start_gather(b, c):
            pltpu.sync_copy(idx_hbm.at[pl.ds(row0 + c * chunk, chunk)], idx[b])
            pltpu.async_copy(table_hbm.at[idx[b]], rows[b], gsem[b])

        def wait_gather(b):
            pltpu.make_async_copy(table_hbm.at[idx[b]], rows[b], gsem[b]).wait()

        def start_write(b, c):
            pltpu.async_copy(rows[b], out_hbm.at[pl.ds(row0 + c * chunk, chunk)], osem[b])

        def wait_write(b):
            pltpu.make_async_copy(rows[b], out_hbm.at[pl.ds(row0, chunk)], osem[b]).wait()

        def compute_seg(b0, b1, lo, hi):
            @plsc.parallel_loop(lo, hi, unroll=2)
            def per_row(r):
                for e in range(vregs_per_row):
                    sl = pl.ds(e * _LANES, _LANES)
                    pv = pe_v[r, sl]
                    rows[b0][r, sl] = rows[b0][r, sl] * coef + pv
                    rows[b1][r, sl] = rows[b1][r, sl] * coef + pv

        start_gather(0, 0)
        start_gather(1, 1)

        def pair_block(i, k):
            b0, b1 = 2 * k, 2 * k + 1
            o0, o1 = 2 - 2 * k, 3 - 2 * k
            p = 2 * i + k
            wait_gather(b0)
            wait_gather(b1)
            compute_seg(b0, b1, seg[0], seg[1])

            def prefetch(o, c):
                wait_write(o)
                start_gather(o, c)

            if k == 0:
                @pl.when(i > 0)
                def _():
                    wait_write(o0)

                start_gather(o0, 2 * (p + 1))
                compute_seg(b0, b1, seg[1], seg[2])

                @pl.when(i > 0)
                def _():
                    wait_write(o1)

                start_gather(o1, 2 * (p + 1) + 1)
            else:
                @pl.when(i < n_outer - 1)
                def _():
                    prefetch(o0, 2 * (p + 1))

                compute_seg(b0, b1, seg[1], seg[2])

                @pl.when(i < n_outer - 1)
                def _():
                    prefetch(o1, 2 * (p + 1) + 1)

            compute_seg(b0, b1, seg[2], seg[3])
            start_write(b0, 2 * p)
            start_write(b1, 2 * p + 1)

        def outer(i, carry):
            pair_block(i, 0)
            pair_block(i, 1)
            return carry

        lax.fori_loop(0, n_outer, outer, 0)
        for b in range(_NBUF):
            wait_write(b)

    return sc_kernel


def kernel(x, table, pos_enc):
    batch, seq_len = x.shape
    vocab, embed = table.shape
    n_rows = batch * seq_len
    xf = x.reshape(n_rows).astype(jnp.int32)
    sc = _make_sc_kernel(n_rows, vocab, embed, seq_len)
    out = sc(xf, table, pos_enc)
    return out.reshape(batch, seq_len, embed)


# speedup vs baseline: 1.0300x; 1.0300x over previous
import functools

import jax
import jax.numpy as jnp
from jax import lax
from jax.experimental import pallas as pl
from jax.experimental.pallas import tpu as pltpu
from jax.experimental.pallas import tpu_sc as plsc

_LANES = 16
_NBUF = 4


def _make_sc_kernel(n_rows, vocab, embed, seq_len):
    n_workers = 32
    assert n_rows % (n_workers * seq_len) == 0
    rows_per_w = n_rows // n_workers
    chunk = seq_len
    n_chunks = rows_per_w // chunk
    n_pairs = n_chunks // 2
    n_outer = n_pairs // 2
    assert n_pairs % 2 == 0 and chunk % 8 == 0
    vregs_per_row = embed // _LANES
    seg = [0, chunk // 3 // 8 * 8, 2 * (chunk // 3) // 8 * 8, chunk]

    mesh = plsc.VectorSubcoreMesh(core_axis_name="c", subcore_axis_name="s")

    @functools.partial(
        pl.kernel,
        mesh=mesh,
        out_type=jax.ShapeDtypeStruct((n_rows, embed), jnp.float32),
        scratch_types=[
            [pltpu.VMEM((chunk,), jnp.int32) for _ in range(_NBUF)],
            [pltpu.VMEM((chunk, embed), jnp.float32) for _ in range(_NBUF)],
            pltpu.VMEM((seq_len, embed), jnp.float32),
            [pltpu.SemaphoreType.DMA for _ in range(_NBUF)],
            [pltpu.SemaphoreType.DMA for _ in range(_NBUF)],
        ],
    )
    def sc_kernel(idx_hbm, table_hbm, pe_hbm, out_hbm, idx, rows, pe_v, gsem, osem):
        wid = lax.axis_index("s") * 2 + lax.axis_index("c")
        row0 = wid * rows_per_w
        coef = jnp.float32(1.0 / (embed**0.5))

        pltpu.sync_copy(pe_hbm, pe_v)

        def start_gather(b, c):
            pltpu.sync_copy(idx_hbm.at[pl.ds(row0 + c * chunk, chunk)], idx[b])
            pltpu.async_copy(table_hbm.at[idx[b]], rows[b], gsem[b])

        def wait_gather(b):
            pltpu.make_async_copy(table_hbm.at[idx[b]], rows[b], gsem[b]).wait()

        def start_write(b, c):
            pltpu.async_copy(rows[b], out_hbm.at[pl.ds(row0 + c * chunk, chunk)], osem[b])

        def wait_write(b):
            pltpu.make_async_copy(rows[b], out_hbm.at[pl.ds(row0, chunk)], osem[b]).wait()

        def compute_seg(b0, b1, lo, hi):
            @plsc.parallel_loop(lo, hi, unroll=1)
            def per_row(r):
                for e in range(vregs_per_row):
                    sl = pl.ds(e * _LANES, _LANES)
                    pv = pe_v[r, sl]
                    rows[b0][r, sl] = rows[b0][r, sl] * coef + pv
                    rows[b1][r, sl] = rows[b1][r, sl] * coef + pv

        start_gather(0, 0)
        start_gather(1, 1)

        def pair_block(i, k):
            b0, b1 = 2 * k, 2 * k + 1
            o0, o1 = 2 - 2 * k, 3 - 2 * k
            p = 2 * i + k
            wait_gather(b0)
            wait_gather(b1)
            compute_seg(b0, b1, seg[0], seg[1])

            def prefetch(o, c):
                wait_write(o)
                start_gather(o, c)

            if k == 0:
                @pl.when(i > 0)
                def _():
                    wait_write(o0)

                start_gather(o0, 2 * (p + 1))
                compute_seg(b0, b1, seg[1], seg[2])

                @pl.when(i > 0)
                def _():
                    wait_write(o1)

                start_gather(o1, 2 * (p + 1) + 1)
            else:
                @pl.when(i < n_outer - 1)
                def _():
                    prefetch(o0, 2 * (p + 1))

                compute_seg(b0, b1, seg[1], seg[2])

                @pl.when(i < n_outer - 1)
                def _():
                    prefetch(o1, 2 * (p + 1) + 1)

            compute_seg(b0, b1, seg[2], seg[3])
            start_write(b0, 2 * p)
            start_write(b1, 2 * p + 1)

        def outer(i, carry):
            pair_block(i, 0)
            pair_block(i, 1)
            return carry

        lax.fori_loop(0, n_outer, outer, 0)
        for b in range(_NBUF):
            wait_write(b)

    return sc_kernel


def kernel(x, table, pos_enc):
    batch, seq_len = x.shape
    vocab, embed = table.shape
    n_rows = batch * seq_len
    xf = x.reshape(n_rows).astype(jnp.int32)
    sc = _make_sc_kernel(n_rows, vocab, embed, seq_len)
    out = sc(xf, table, pos_enc)
    return out.reshape(batch, seq_len, embed)
